# Initial kernel scaffold; baseline (speedup 1.0000x reference)
import jax
import jax.numpy as jnp
from jax.experimental import pallas as pl


def kernel(logits, k, p, q):
    raise NotImplementedError("write your pallas kernel here")


# speedup vs baseline: 4.8653x; 4.8653x over previous
import jax
import jax.numpy as jnp
from jax.experimental import pallas as pl

_C = 72
_R = 8


def _sampler_kernel(logits_ref, k_ref, p_ref, q_ref, out_ref):
    R, V = logits_ref.shape
    C = _C
    iota_v = jax.lax.broadcasted_iota(jnp.int32, (R, V), 1)
    iota_c = jax.lax.broadcasted_iota(jnp.int32, (R, C), 1)
    neg_inf = jnp.float32(-jnp.inf)

    w0 = logits_ref[...]
    qb = q_ref[...]

    vals0 = jnp.full((R, C), neg_inf, jnp.float32)
    idxs0 = jnp.zeros((R, C), jnp.int32)
    qvs0 = jnp.ones((R, C), jnp.float32)

    def body(t, carry):
        w, vals, idxs, qvs = carry
        m = jnp.max(w, axis=1)
        eqm = w == m[:, None]
        mi = jnp.min(jnp.where(eqm, iota_v, V), axis=1)
        sel = iota_v == mi[:, None]
        qv = jnp.sum(jnp.where(sel, qb, 0.0), axis=1)
        w = jnp.where(sel, neg_inf, w)
        slot = iota_c == t
        vals = jnp.where(slot, m[:, None], vals)
        idxs = jnp.where(slot, mi[:, None], idxs)
        qvs = jnp.where(slot, qv[:, None], qvs)
        return w, vals, idxs, qvs

    wf, vals, idxs, qvs = jax.lax.fori_loop(0, C, body, (w0, vals0, idxs0, qvs0))

    kb = k_ref[...]
    pb = p_ref[...]

    thr = jnp.sum(jnp.where(iota_c == (kb - 1), vals, 0.0), axis=1, keepdims=True)
    sv = vals >= thr

    m0 = jnp.max(vals, axis=1, keepdims=True)
    e1 = jnp.where(sv, jnp.exp(vals - m0), 0.0)
    z1 = jnp.sum(e1, axis=1, keepdims=True)
    pr = e1 / z1

    vj = vals[:, :, None]
    vl = vals[:, None, :]
    ij = idxs[:, :, None]
    il = idxs[:, None, :]
    leq = (vl < vj) | ((vl == vj) & (il <= ij))
    csum = jnp.sum(jnp.where(leq, pr[:, None, :], 0.0), axis=2)

    maxidx = jnp.max(jnp.where(vals == m0, idxs, -1), axis=1, keepdims=True)
    is_last = (vals == m0) & (idxs == maxidx)
    drop = (csum <= (1.0 - pb)) & jnp.logical_not(is_last)
    keep = sv & jnp.logical_not(drop)

    m2 = jnp.max(jnp.where(keep, vals, neg_inf), axis=1, keepdims=True)
    e2 = jnp.where(keep, jnp.exp(vals - m2), 0.0)
    z2 = jnp.sum(e2, axis=1, keepdims=True)
    r = jnp.where(keep, (e2 / z2) / qvs, -1.0)
    rmax = jnp.max(r, axis=1, keepdims=True)
    ans = jnp.min(jnp.where(r == rmax, idxs, V), axis=1)

    nz1 = jnp.min(
        jnp.where((qb == 0.0) & (wf != neg_inf), iota_v, V), axis=1
    )
    nz2 = jnp.min(
        jnp.where((qvs == 0.0) & jnp.logical_not(keep), idxs, V), axis=1
    )
    nanidx = jnp.minimum(nz1, nz2)
    ans = jnp.where(nanidx < V, nanidx, ans)
    out_ref[...] = ans[:, None]


@jax.jit
def _run(logits, k, p, q):
    B, V = logits.shape
    k2 = k.reshape(B, 1).astype(jnp.int32)
    p2 = p.reshape(B, 1).astype(jnp.float32)
    out = pl.pallas_call(
        _sampler_kernel,
        grid=(B // _R,),
        in_specs=[
            pl.BlockSpec((_R, V), lambda i: (i, 0)),
            pl.BlockSpec((_R, 1), lambda i: (i, 0)),
            pl.BlockSpec((_R, 1), lambda i: (i, 0)),
            pl.BlockSpec((_R, V), lambda i: (i, 0)),
        ],
        out_specs=pl.BlockSpec((_R, 1), lambda i: (i, 0)),
        out_shape=jax.ShapeDtypeStruct((B, 1), jnp.int32),
    )(logits, k2, p2, q)
    return out.reshape(-1)


def kernel(logits, k, p, q):
    return _run(logits, k, p, q)


# speedup vs baseline: 6.2491x; 1.2844x over previous
import jax
import jax.numpy as jnp
from jax.experimental import pallas as pl

_C = 72
_R = 8


def _sampler_kernel(logits_ref, k_ref, p_ref, q_ref, out_ref):
    R, V = logits_ref.shape
    C = _C
    iota_v = jax.lax.broadcasted_iota(jnp.int32, (R, V), 1)
    iota_c = jax.lax.broadcasted_iota(jnp.int32, (R, C), 1)
    neg_inf = jnp.float32(-jnp.inf)

    w0 = logits_ref[...]
    qb = q_ref[...]

    vals0 = jnp.full((R, C), neg_inf, jnp.float32)
    idxs0 = jnp.zeros((R, C), jnp.int32)

    def body(t, carry):
        w, vals, idxs = carry
        m = jnp.max(w, axis=1)
        eqm = w == m[:, None]
        mi = jnp.min(jnp.where(eqm, iota_v, V), axis=1)
        w = jnp.where(iota_v == mi[:, None], neg_inf, w)
        slot = iota_c == t
        vals = jnp.where(slot, m[:, None], vals)
        idxs = jnp.where(slot, mi[:, None], idxs)
        return w, vals, idxs

    _, vals, idxs = jax.lax.fori_loop(0, C, body, (w0, vals0, idxs0))

    kb = k_ref[...]
    pb = p_ref[...]

    thr = jnp.sum(jnp.where(iota_c == (kb - 1), vals, 0.0), axis=1, keepdims=True)
    sv = vals >= thr

    m0 = jnp.max(vals, axis=1, keepdims=True)
    e1 = jnp.where(sv, jnp.exp(vals - m0), 0.0)
    z1 = jnp.sum(e1, axis=1, keepdims=True)
    pr = e1 / z1

    vj = vals[:, :, None]
    vl = vals[:, None, :]
    ij = idxs[:, :, None]
    il = idxs[:, None, :]
    leq = (vl < vj) | ((vl == vj) & (il <= ij))
    csum = jnp.sum(jnp.where(leq, pr[:, None, :], 0.0), axis=2)

    maxidx = jnp.max(jnp.where(vals == m0, idxs, -1), axis=1, keepdims=True)
    is_last = (vals == m0) & (idxs == maxidx)
    drop = (csum <= (1.0 - pb)) & jnp.logical_not(is_last)
    keep = sv & jnp.logical_not(drop)

    pos_inf = jnp.float32(jnp.inf)
    vB = jnp.min(jnp.where(keep, vals, pos_inf), axis=1, keepdims=True)
    iB = jnp.min(jnp.where(keep & (vals == vB), idxs, V), axis=1, keepdims=True)
    m2 = jnp.max(jnp.where(keep, vals, neg_inf), axis=1, keepdims=True)
    z2 = jnp.sum(jnp.where(keep, jnp.exp(vals - m2), 0.0), axis=1, keepdims=True)

    keptf = (w0 > vB) | ((w0 == vB) & (iota_v >= iB))
    t_full = jnp.where(keptf, (jnp.exp(w0 - m2) / z2) / qb, -1.0)
    rmax = jnp.max(t_full, axis=1, keepdims=True)
    ans = jnp.min(jnp.where(t_full == rmax, iota_v, V), axis=1)

    nanidx = jnp.min(
        jnp.where((qb == 0.0) & jnp.logical_not(keptf), iota_v, V), axis=1
    )
    ans = jnp.where(nanidx < V, nanidx, ans)
    out_ref[...] = ans[:, None]


@jax.jit
def _run(logits, k, p, q):
    B, V = logits.shape
    k2 = k.reshape(B, 1).astype(jnp.int32)
    p2 = p.reshape(B, 1).astype(jnp.float32)
    out = pl.pallas_call(
        _sampler_kernel,
        grid=(B // _R,),
        in_specs=[
            pl.BlockSpec((_R, V), lambda i: (i, 0)),
            pl.BlockSpec((_R, 1), lambda i: (i, 0)),
            pl.BlockSpec((_R, 1), lambda i: (i, 0)),
            pl.BlockSpec((_R, V), lambda i: (i, 0)),
        ],
        out_specs=pl.BlockSpec((_R, 1), lambda i: (i, 0)),
        out_shape=jax.ShapeDtypeStruct((B, 1), jnp.int32),
    )(logits, k2, p2, q)
    return out.reshape(-1)


def kernel(logits, k, p, q):
    return _run(logits, k, p, q)


# speedup vs baseline: 8.9694x; 1.4353x over previous
import jax
import jax.numpy as jnp
from jax.experimental import pallas as pl

_C = 68
_R = 16


def _sampler_kernel(logits_ref, k_ref, p_ref, q_ref, out_ref):
    R, V = logits_ref.shape
    C = _C
    iota_v = jax.lax.broadcasted_iota(jnp.int32, (R, V), 1)
    iota_c = jax.lax.broadcasted_iota(jnp.int32, (R, C), 1)
    neg_inf = jnp.float32(-jnp.inf)

    w0 = logits_ref[...]
    qb = q_ref[...]

    vals0 = jnp.full((R, C), neg_inf, jnp.float32)
    idxs0 = jnp.zeros((R, C), jnp.int32)

    def body(t, carry):
        w, vals, idxs = carry
        m = jnp.max(w, axis=1)
        eqm = w == m[:, None]
        mi = jnp.min(jnp.where(eqm, iota_v, V), axis=1)
        w = jnp.where(iota_v == mi[:, None], neg_inf, w)
        slot = iota_c == t
        vals = jnp.where(slot, m[:, None], vals)
        idxs = jnp.where(slot, mi[:, None], idxs)
        return w, vals, idxs

    _, vals, idxs = jax.lax.fori_loop(0, C, body, (w0, vals0, idxs0))

    kb = k_ref[...]
    pb = p_ref[...]

    thr = jnp.sum(jnp.where(iota_c == (kb - 1), vals, 0.0), axis=1, keepdims=True)
    sv = vals >= thr

    m0 = jnp.max(vals, axis=1, keepdims=True)
    e1 = jnp.where(sv, jnp.exp(vals - m0), 0.0)
    z1 = jnp.sum(e1, axis=1, keepdims=True)
    pr = e1 / z1

    vj = vals[:, :, None]
    vl = vals[:, None, :]
    ij = idxs[:, :, None]
    il = idxs[:, None, :]
    leq = (vl < vj) | ((vl == vj) & (il <= ij))
    csum = jnp.sum(jnp.where(leq, pr[:, None, :], 0.0), axis=2)

    maxidx = jnp.max(jnp.where(vals == m0, idxs, -1), axis=1, keepdims=True)
    is_last = (vals == m0) & (idxs == maxidx)
    drop = (csum <= (1.0 - pb)) & jnp.logical_not(is_last)
    keep = sv & jnp.logical_not(drop)

    pos_inf = jnp.float32(jnp.inf)
    vB = jnp.min(jnp.where(keep, vals, pos_inf), axis=1, keepdims=True)
    iB = jnp.min(jnp.where(keep & (vals == vB), idxs, V), axis=1, keepdims=True)
    m2 = jnp.max(jnp.where(keep, vals, neg_inf), axis=1, keepdims=True)
    z2 = jnp.sum(jnp.where(keep, jnp.exp(vals - m2), 0.0), axis=1, keepdims=True)

    keptf = (w0 > vB) | ((w0 == vB) & (iota_v >= iB))
    t_full = jnp.where(keptf, (jnp.exp(w0 - m2) / z2) / qb, -1.0)
    rmax = jnp.max(t_full, axis=1, keepdims=True)
    ans = jnp.min(jnp.where(t_full == rmax, iota_v, V), axis=1)

    nanidx = jnp.min(
        jnp.where((qb == 0.0) & jnp.logical_not(keptf), iota_v, V), axis=1
    )
    ans = jnp.where(nanidx < V, nanidx, ans)
    out_ref[...] = ans[:, None]


@jax.jit
def _run(logits, k, p, q):
    B, V = logits.shape
    k2 = k.reshape(B, 1).astype(jnp.int32)
    p2 = p.reshape(B, 1).astype(jnp.float32)
    out = pl.pallas_call(
        _sampler_kernel,
        grid=(B // _R,),
        in_specs=[
            pl.BlockSpec((_R, V), lambda i: (i, 0)),
            pl.BlockSpec((_R, 1), lambda i: (i, 0)),
            pl.BlockSpec((_R, 1), lambda i: (i, 0)),
            pl.BlockSpec((_R, V), lambda i: (i, 0)),
        ],
        out_specs=pl.BlockSpec((_R, 1), lambda i: (i, 0)),
        out_shape=jax.ShapeDtypeStruct((B, 1), jnp.int32),
    )(logits, k2, p2, q)
    return out.reshape(-1)


def kernel(logits, k, p, q):
    return _run(logits, k, p, q)
